# Initial kernel scaffold; baseline (speedup 1.0000x reference)
import jax
import jax.numpy as jnp
from jax.experimental import pallas as pl


def kernel(input, mask, init_mask):
    raise NotImplementedError("write your pallas kernel here")


# speedup vs baseline: 15.9742x; 15.9742x over previous
import functools

import jax
import jax.numpy as jnp
from jax.experimental import pallas as pl

_B, _N, _F, _K = 32, 8192, 16, 1024
_EPS = 1e-10
_IMIN = -2147483648


def _min_kernel(x_ref, o_ref):
    b = pl.program_id(0)
    m = jnp.full((1, 1), jnp.min(jnp.abs(x_ref[0])), jnp.float32)

    @pl.when(b == 0)
    def _():
        o_ref[:, :] = m

    @pl.when(b != 0)
    def _():
        o_ref[:, :] = jnp.minimum(o_ref[:, :], m)


def _select_kernel(xt_ref, m0t_ref, minv_ref, out_ref, mask_ref):
    x = xt_ref[0]
    m0 = m0t_ref[0]
    v = (x + (minv_ref[:, :] + _EPS)) * m0
    bits = jax.lax.bitcast_convert_type(v, jnp.int32)
    keys = jnp.where(bits < 0, bits ^ jnp.int32(0x7FFFFFFF), bits)

    def vbody(i, p):
        cand = p | jax.lax.shift_left(jnp.int32(1), 31 - i)
        cnt = jnp.sum((keys >= (cand ^ jnp.int32(_IMIN))).astype(jnp.int32),
                      axis=1, keepdims=True)
        return jnp.where(cnt >= _K, cand, p)

    p = jax.lax.fori_loop(0, 32, vbody, jnp.zeros((_F, 1), jnp.int32))
    tkey = p ^ jnp.int32(_IMIN)

    gt = keys > tkey
    eq = keys == tkey
    g0 = jnp.sum(gt.astype(jnp.int32), axis=1, keepdims=True)
    iota = jax.lax.broadcasted_iota(jnp.int32, (_F, _N), 1)

    def ibody(i, p2):
        cand = p2 | jax.lax.shift_left(jnp.int32(1), 12 - i)
        cnt = g0 + jnp.sum((eq & (iota <= cand)).astype(jnp.int32), axis=1,
                           keepdims=True)
        return jnp.where(cnt < _K, cand, p2)

    p2 = jax.lax.fori_loop(0, 13, ibody, jnp.zeros((_F, 1), jnp.int32))
    gp = g0 + jnp.sum((eq & (iota <= p2)).astype(jnp.int32), axis=1,
                      keepdims=True)
    jstar = p2 + (gp < _K).astype(jnp.int32)

    sel = gt | (eq & (iota <= jstar))
    maskf = jnp.any(sel, axis=0, keepdims=True).astype(jnp.float32)
    mask_ref[0] = maskf
    out_ref[0] = x * maskf


@jax.jit
def kernel(input, mask, init_mask):
    del mask
    xt = jnp.transpose(input, (0, 2, 1))
    m0t = jnp.transpose(init_mask, (0, 2, 1))

    minv = pl.pallas_call(
        _min_kernel,
        grid=(_B,),
        in_specs=[pl.BlockSpec((1, _F, _N), lambda b: (b, 0, 0))],
        out_specs=pl.BlockSpec((1, 1), lambda b: (0, 0)),
        out_shape=jax.ShapeDtypeStruct((1, 1), jnp.float32),
    )(xt)

    out_t, mask_t = pl.pallas_call(
        _select_kernel,
        grid=(_B,),
        in_specs=[
            pl.BlockSpec((1, _F, _N), lambda b: (b, 0, 0)),
            pl.BlockSpec((1, 1, _N), lambda b: (b, 0, 0)),
            pl.BlockSpec((1, 1), lambda b: (0, 0)),
        ],
        out_specs=[
            pl.BlockSpec((1, _F, _N), lambda b: (b, 0, 0)),
            pl.BlockSpec((1, 1, _N), lambda b: (b, 0, 0)),
        ],
        out_shape=[
            jax.ShapeDtypeStruct((_B, _F, _N), jnp.float32),
            jax.ShapeDtypeStruct((_B, 1, _N), jnp.float32),
        ],
    )(xt, m0t, minv)

    updated_mask = jnp.reshape(mask_t, (_B, _N, 1))
    masked_out = jnp.transpose(out_t, (0, 2, 1))
    return (updated_mask, masked_out)
